# baseline (speedup 1.0000x reference)
import functools

import jax
import jax.numpy as jnp
from jax import lax
from jax.experimental import pallas as pl
from jax.experimental.pallas import tpu as pltpu
from jax.experimental.pallas import tpu_sc as plsc

D, H, E, TOPK = 1024, 4096, 8, 2
T = 2048
A = TOPK * T
BLK = 512
NUM_BLOCKS = A // BLK + E
NP = NUM_BLOCKS * BLK
NH = 2
HT = H // NH
NEG = -1e30

NTR = 32
APT = A // NTR
NTC = 32
TPW = T // NTC


def _gating_body(x_ref, wg_ref, bg_ref, idx_ref, g_ref, aux_ref):
    x = x_ref[...]
    logits = jnp.dot(x, wg_ref[...], preferred_element_type=jnp.float32)
    logits = logits + bg_ref[...]
    col = lax.broadcasted_iota(jnp.int32, logits.shape, 1)
    valid = col < E
    lg = jnp.where(valid, logits, NEG)
    m1 = jnp.max(lg, axis=1, keepdims=True)
    i1 = jnp.min(jnp.where(lg == m1, col, 127), axis=1, keepdims=True)
    lg2 = jnp.where(col == i1, NEG, lg)
    m2 = jnp.max(lg2, axis=1, keepdims=True)
    i2 = jnp.min(jnp.where(lg2 == m2, col, 127), axis=1, keepdims=True)
    e2 = jnp.exp(m2 - m1)
    g1 = 1.0 / (1.0 + e2)
    g2 = e2 / (1.0 + e2)
    idx_ref[...] = jnp.where(col == 0, i1, jnp.where(col == 1, i2, 0))
    g_ref[...] = jnp.where(col == 0, g1, jnp.where(col == 1, g2, 0.0))
    p = jnp.where(valid, jnp.exp(lg - m1), 0.0)
    p = p / jnp.sum(p, axis=1, keepdims=True)
    imp = jnp.sum(p, axis=0, keepdims=True)
    vrow = col[0:1, :] < E
    mean = jnp.sum(jnp.where(vrow, imp, 0.0)) / E
    var = jnp.sum(jnp.where(vrow, (imp - mean) ** 2, 0.0)) / (E - 1)
    aux_ref[...] = (var / (mean + 1e-8) ** 2).reshape(1, 1)


def _gating(x_flat, wg_pad, bg_pad):
    return pl.pallas_call(
        _gating_body,
        out_shape=(
            jax.ShapeDtypeStruct((T, 128), jnp.int32),
            jax.ShapeDtypeStruct((T, 128), jnp.float32),
            jax.ShapeDtypeStruct((1, 1), jnp.float32),
        ),
    )(x_flat, wg_pad, bg_pad)


def _routing_body(ef_ref, x_ref, xs_ref, pos2_ref, eo_ref, sz_ref,
                  ev_v, lr_v, run_v, base_v, tot_v, fb_v, ends_v,
                  eo_v, sz_v, posq_v, tokq_v, row_v, sem):
    wid = lax.axis_index("s") * 2 + lax.axis_index("c")
    r0 = wid * (APT // 64)
    a0 = wid * APT
    pltpu.sync_copy(ef_ref, ev_v)

    lane = lax.iota(jnp.int32, 16)

    def _count_row(t, carry):
        tot, prior = carry
        rc = jnp.zeros((16,), jnp.int32)
        for l in range(4):
            ev_c = ev_v[pl.ds(t * 64 + l * 16, 16)]
            for e in range(E):
                m = ev_c == e
                pc = plsc.all_reduce_population_count(m)
                rc = rc + jnp.where(lane == e, pc, 0)
        tm = jnp.broadcast_to(t < r0, (16,))
        return tot + rc, prior + jnp.where(tm, rc, 0)

    tot, prior = lax.fori_loop(
        0, A // 64, _count_row,
        (jnp.zeros((16,), jnp.int32), jnp.zeros((16,), jnp.int32)))

    run = jnp.zeros((16,), jnp.int32)
    run_v[...] = run
    for c in range(APT // 16):
        ev_c = ev_v[pl.ds(a0 + c * 16, 16)]
        base_c = plsc.load_gather(run_v, [ev_c])
        rank_c = jnp.zeros((16,), jnp.int32)
        for e in range(E):
            m = ev_c == e
            incl = plsc.cumsum(jnp.where(m, 1, 0))
            rank_c = jnp.where(m, incl - 1, rank_c)
            pc = plsc.all_reduce_population_count(m)
            run = run + jnp.where(lane == e, pc, 0)
        lr_v[pl.ds(c * 16, 16)] = base_c + rank_c
        run_v[...] = run

    bpe = (tot + (BLK - 1)) // BLK
    fb_incl = plsc.cumsum(bpe)
    first_block = fb_incl - bpe
    tot_v[...] = tot
    fb_v[...] = first_block
    ends_v[...] = fb_incl
    base_v[...] = first_block * BLK + prior

    for r in range(APT // 64):
        for l in range(4):
            c = r * 4 + l
            ev_c = ev_v[pl.ds(a0 + c * 16, 16)]
            lr_c = lr_v[pl.ds(c * 16, 16)]
            pos_c = plsc.load_gather(base_v, [ev_c]) + lr_c
            posq_v[r, pl.ds(l * 16, 16)] = jnp.minimum(
                jnp.maximum(pos_c, 0), NP - 1)
            a_c = (a0 + c * 16) + lax.iota(jnp.int32, 16)
            tokq_v[r, pl.ds(l * 16, 16)] = a_c // TOPK
    pltpu.sync_copy(posq_v, pos2_ref.at[pl.ds(r0, APT // 64)])

    for r in range(APT // 64):
        pltpu.async_copy(x_ref.at[tokq_v.at[r]], row_v, sem).wait()
        pltpu.async_copy(row_v, xs_ref.at[posq_v.at[r]], sem).wait()

    @pl.when(wid == 0)
    def _():
        ib = lax.iota(jnp.int32, 16)
        eo_v[...] = jnp.zeros((16,), jnp.int32)
        plsc.store_scatter(eo_v, [jnp.minimum(first_block, 15)],
                           lane + 1, mask=bpe > 0)
        eo = jnp.minimum(jnp.maximum(plsc.cummax(eo_v[...]) - 1, 0), E - 1)
        cnt_e = plsc.load_gather(tot_v, [eo])
        fb_e = plsc.load_gather(fb_v, [eo])
        szv = cnt_e - (ib - fb_e) * BLK
        szv = jnp.minimum(jnp.maximum(szv, 0), BLK)
        eo_v[...] = eo
        sz_v[...] = szv
        pltpu.sync_copy(eo_v, eo_ref)
        pltpu.sync_copy(sz_v, sz_ref)


def _routing(eflat, x_flat):
    mesh = plsc.VectorSubcoreMesh(core_axis_name="c", subcore_axis_name="s")
    f = pl.kernel(
        _routing_body,
        out_type=(
            jax.ShapeDtypeStruct((NP, D), jnp.float32),
            jax.ShapeDtypeStruct((A // 64, 64), jnp.int32),
            jax.ShapeDtypeStruct((16,), jnp.int32),
            jax.ShapeDtypeStruct((16,), jnp.int32),
        ),
        mesh=mesh,
        compiler_params=pltpu.CompilerParams(needs_layout_passes=False),
        scratch_types=[
            pltpu.VMEM((A,), jnp.int32),
            pltpu.VMEM((APT,), jnp.int32),
            pltpu.VMEM((16,), jnp.int32),
            pltpu.VMEM((16,), jnp.int32),
            pltpu.VMEM((16,), jnp.int32),
            pltpu.VMEM((16,), jnp.int32),
            pltpu.VMEM((16,), jnp.int32),
            pltpu.VMEM((16,), jnp.int32),
            pltpu.VMEM((16,), jnp.int32),
            pltpu.VMEM((APT // 64, 64), jnp.int32),
            pltpu.VMEM((APT // 64, 64), jnp.int32),
            pltpu.VMEM((64, D), jnp.float32),
            pltpu.SemaphoreType.DMA,
        ],
    )
    return f(eflat, x_flat)


def _combine_body(y_ref, pos2_ref, g_ref, out_ref,
                  posb_v, gb_v, pair_v, ob_v, sem):
    wid = lax.axis_index("s") * 2 + lax.axis_index("c")
    r0 = wid * 2
    pltpu.sync_copy(pos2_ref.at[pl.ds(r0, 2)], posb_v)
    pltpu.sync_copy(g_ref.at[pl.ds(r0, 2)], gb_v)
    for r in range(2):
        pltpu.async_copy(y_ref.at[posb_v.at[r]], pair_v, sem).wait()

        def body(i, _):
            rfull = jnp.full((16,), r, jnp.int32)
            g0 = plsc.load_gather(
                gb_v, [rfull, jnp.broadcast_to(2 * i, (16,))])
            g1 = plsc.load_gather(
                gb_v, [rfull, jnp.broadcast_to(2 * i + 1, (16,))])
            for ch in range(D // 16):
                va = pair_v[2 * i, pl.ds(ch * 16, 16)]
                vb = pair_v[2 * i + 1, pl.ds(ch * 16, 16)]
                ob_v[i, pl.ds(ch * 16, 16)] = va * g0 + vb * g1
            return 0

        lax.fori_loop(0, 32, body, 0)
        pltpu.sync_copy(ob_v, out_ref.at[pl.ds(wid * TPW + r * 32, 32)])


def _combine(y, pos2, gflat2d):
    mesh = plsc.VectorSubcoreMesh(core_axis_name="c", subcore_axis_name="s")
    f = pl.kernel(
        _combine_body,
        out_type=jax.ShapeDtypeStruct((T, D), jnp.float32),
        mesh=mesh,
        compiler_params=pltpu.CompilerParams(needs_layout_passes=False),
        scratch_types=[
            pltpu.VMEM((2, 64), jnp.int32),
            pltpu.VMEM((2, 64), jnp.float32),
            pltpu.VMEM((64, D), jnp.float32),
            pltpu.VMEM((32, D), jnp.float32),
            pltpu.SemaphoreType.DMA,
        ],
    )
    return f(y, pos2, gflat2d)


def _ffn_body(e_of_ref, sz_ref, xs_ref, w1_ref, b1_ref, w2_ref, b2_ref,
              y_ref, acc_ref):
    j = pl.program_id(1)

    @pl.when(j == 0)
    def _():
        acc_ref[...] = jnp.zeros_like(acc_ref)

    @pl.when(sz_ref[pl.program_id(0)] > 0)
    def _():
        h = jnp.dot(xs_ref[...], w1_ref[0], preferred_element_type=jnp.float32)
        h = jnp.maximum(h + b1_ref[0], 0.0)
        acc_ref[...] += jnp.dot(h, w2_ref[0], preferred_element_type=jnp.float32)

    @pl.when(j == NH - 1)
    def _():
        y_ref[...] = acc_ref[...] + b2_ref[0]


def _serp(i, j):
    return jnp.where(i % 2 == 0, j, NH - 1 - j)


def _ffn(e_of, sz, xs, w1, b1, w2, b2):
    grid_spec = pltpu.PrefetchScalarGridSpec(
        num_scalar_prefetch=2,
        grid=(NUM_BLOCKS, NH),
        in_specs=[
            pl.BlockSpec((BLK, D), lambda i, j, eo, sz: (i, 0)),
            pl.BlockSpec((1, D, HT), lambda i, j, eo, sz: (eo[i], 0, _serp(i, j))),
            pl.BlockSpec((1, 1, HT), lambda i, j, eo, sz: (eo[i], 0, _serp(i, j))),
            pl.BlockSpec((1, HT, D), lambda i, j, eo, sz: (eo[i], _serp(i, j), 0)),
            pl.BlockSpec((1, 1, D), lambda i, j, eo, sz: (eo[i], 0, 0)),
        ],
        out_specs=pl.BlockSpec((BLK, D), lambda i, j, eo, sz: (i, 0)),
        scratch_shapes=[pltpu.VMEM((BLK, D), jnp.float32)],
    )
    return pl.pallas_call(
        _ffn_body,
        grid_spec=grid_spec,
        out_shape=jax.ShapeDtypeStruct((NP, D), jnp.float32),
        compiler_params=pltpu.CompilerParams(
            dimension_semantics=("arbitrary", "arbitrary"),
            vmem_limit_bytes=128 * 1024 * 1024,
        ),
    )(e_of, sz, xs, w1, b1, w2, b2)


def kernel(x, Wg, bg, W1, b1, W2, b2):
    B, S, d = x.shape
    x_flat = x.reshape(T, d)
    wg_pad = jnp.pad(Wg, ((0, 0), (0, 128 - E)))
    bg_pad = jnp.pad(bg, (0, 128 - E)).reshape(1, 128)

    idx128, g128, aux = _gating(x_flat, wg_pad, bg_pad)
    aux = aux[0, 0]
    eflat = idx128[:, :TOPK].reshape(A)
    gflat2d = g128[:, :TOPK].reshape(A // 64, 64)

    xs, pos2, e_of, sz = _routing(eflat, x_flat)
    y = _ffn(e_of, sz, xs, W1, b1.reshape(E, 1, H), W2, b2.reshape(E, 1, D))
    out = _combine(y, pos2, gflat2d).reshape(B, S, d)
    return out, aux


# speedup vs baseline: 1.0099x; 1.0099x over previous
import functools

import jax
import jax.numpy as jnp
from jax import lax
from jax.experimental import pallas as pl
from jax.experimental.pallas import tpu as pltpu
from jax.experimental.pallas import tpu_sc as plsc

D, H, E, TOPK = 1024, 4096, 8, 2
T = 2048
A = TOPK * T
BLK = 512
NUM_BLOCKS = A // BLK + E
NP = NUM_BLOCKS * BLK
NH = 2
HT = H // NH
NEG = -1e30

NTR = 32
APT = A // NTR
NTC = 32
TPW = T // NTC


def _gating_body(x_ref, wg_ref, bg_ref, idx_ref, g_ref, aux_ref):
    x = x_ref[...]
    logits = jnp.dot(x, wg_ref[...], preferred_element_type=jnp.float32)
    logits = logits + bg_ref[...]
    col = lax.broadcasted_iota(jnp.int32, logits.shape, 1)
    valid = col < E
    lg = jnp.where(valid, logits, NEG)
    m1 = jnp.max(lg, axis=1, keepdims=True)
    i1 = jnp.min(jnp.where(lg == m1, col, 127), axis=1, keepdims=True)
    lg2 = jnp.where(col == i1, NEG, lg)
    m2 = jnp.max(lg2, axis=1, keepdims=True)
    i2 = jnp.min(jnp.where(lg2 == m2, col, 127), axis=1, keepdims=True)
    e2 = jnp.exp(m2 - m1)
    g1 = 1.0 / (1.0 + e2)
    g2 = e2 / (1.0 + e2)
    idx_ref[...] = jnp.where(col == 0, i1, jnp.where(col == 1, i2, 0))
    g_ref[...] = jnp.where(col == 0, g1, jnp.where(col == 1, g2, 0.0))
    p = jnp.where(valid, jnp.exp(lg - m1), 0.0)
    p = p / jnp.sum(p, axis=1, keepdims=True)
    imp = jnp.sum(p, axis=0, keepdims=True)
    vrow = col[0:1, :] < E
    mean = jnp.sum(jnp.where(vrow, imp, 0.0)) / E
    var = jnp.sum(jnp.where(vrow, (imp - mean) ** 2, 0.0)) / (E - 1)
    aux_ref[...] = (var / (mean + 1e-8) ** 2).reshape(1, 1)


def _gating(x_flat, wg_pad, bg_pad):
    return pl.pallas_call(
        _gating_body,
        out_shape=(
            jax.ShapeDtypeStruct((T, 128), jnp.int32),
            jax.ShapeDtypeStruct((T, 128), jnp.float32),
            jax.ShapeDtypeStruct((1, 1), jnp.float32),
        ),
    )(x_flat, wg_pad, bg_pad)


def _routing_body(ef_ref, x_ref, xs_ref, pos2_ref, eo_ref, sz_ref,
                  ev_v, lr_v, run_v, base_v, tot_v, fb_v, ends_v,
                  eo_v, sz_v, posq_v, tokq_v, row_v, sem):
    wid = lax.axis_index("s") * 2 + lax.axis_index("c")
    r0 = wid * (APT // 64)
    a0 = wid * APT
    pltpu.sync_copy(ef_ref, ev_v)

    lane = lax.iota(jnp.int32, 16)

    def _count_row(t, carry):
        tot, prior = carry
        rc = jnp.zeros((16,), jnp.int32)
        for l in range(4):
            ev_c = ev_v[pl.ds(t * 64 + l * 16, 16)]
            for e in range(E):
                m = ev_c == e
                pc = plsc.all_reduce_population_count(m)
                rc = rc + jnp.where(lane == e, pc, 0)
        tm = jnp.broadcast_to(t < r0, (16,))
        return tot + rc, prior + jnp.where(tm, rc, 0)

    tot, prior = lax.fori_loop(
        0, A // 64, _count_row,
        (jnp.zeros((16,), jnp.int32), jnp.zeros((16,), jnp.int32)))

    run = jnp.zeros((16,), jnp.int32)
    run_v[...] = run
    for c in range(APT // 16):
        ev_c = ev_v[pl.ds(a0 + c * 16, 16)]
        base_c = plsc.load_gather(run_v, [ev_c])
        rank_c = jnp.zeros((16,), jnp.int32)
        for e in range(E):
            m = ev_c == e
            incl = plsc.cumsum(jnp.where(m, 1, 0))
            rank_c = jnp.where(m, incl - 1, rank_c)
            pc = plsc.all_reduce_population_count(m)
            run = run + jnp.where(lane == e, pc, 0)
        lr_v[pl.ds(c * 16, 16)] = base_c + rank_c
        run_v[...] = run

    bpe = (tot + (BLK - 1)) // BLK
    fb_incl = plsc.cumsum(bpe)
    first_block = fb_incl - bpe
    tot_v[...] = tot
    fb_v[...] = first_block
    ends_v[...] = fb_incl
    base_v[...] = first_block * BLK + prior

    for r in range(APT // 64):
        for l in range(4):
            c = r * 4 + l
            ev_c = ev_v[pl.ds(a0 + c * 16, 16)]
            lr_c = lr_v[pl.ds(c * 16, 16)]
            pos_c = plsc.load_gather(base_v, [ev_c]) + lr_c
            posq_v[r, pl.ds(l * 16, 16)] = jnp.minimum(
                jnp.maximum(pos_c, 0), NP - 1)
            a_c = (a0 + c * 16) + lax.iota(jnp.int32, 16)
            tokq_v[r, pl.ds(l * 16, 16)] = a_c // TOPK
    pltpu.sync_copy(posq_v, pos2_ref.at[pl.ds(r0, APT // 64)])

    for r in range(APT // 64):
        pltpu.async_copy(x_ref.at[tokq_v.at[r]], row_v, sem).wait()
        pltpu.async_copy(row_v, xs_ref.at[posq_v.at[r]], sem).wait()

    @pl.when(wid == 0)
    def _():
        ib = lax.iota(jnp.int32, 16)
        eo_v[...] = jnp.zeros((16,), jnp.int32)
        plsc.store_scatter(eo_v, [jnp.minimum(first_block, 15)],
                           lane + 1, mask=bpe > 0)
        eo = jnp.minimum(jnp.maximum(plsc.cummax(eo_v[...]) - 1, 0), E - 1)
        cnt_e = plsc.load_gather(tot_v, [eo])
        fb_e = plsc.load_gather(fb_v, [eo])
        szv = cnt_e - (ib - fb_e) * BLK
        szv = jnp.minimum(jnp.maximum(szv, 0), BLK)
        eo_v[...] = eo
        sz_v[...] = szv
        pltpu.sync_copy(eo_v, eo_ref)
        pltpu.sync_copy(sz_v, sz_ref)


def _routing(eflat, x_flat):
    mesh = plsc.VectorSubcoreMesh(core_axis_name="c", subcore_axis_name="s")
    f = pl.kernel(
        _routing_body,
        out_type=(
            jax.ShapeDtypeStruct((NP, D), jnp.float32),
            jax.ShapeDtypeStruct((A // 64, 64), jnp.int32),
            jax.ShapeDtypeStruct((16,), jnp.int32),
            jax.ShapeDtypeStruct((16,), jnp.int32),
        ),
        mesh=mesh,
        compiler_params=pltpu.CompilerParams(needs_layout_passes=False),
        scratch_types=[
            pltpu.VMEM((A,), jnp.int32),
            pltpu.VMEM((APT,), jnp.int32),
            pltpu.VMEM((16,), jnp.int32),
            pltpu.VMEM((16,), jnp.int32),
            pltpu.VMEM((16,), jnp.int32),
            pltpu.VMEM((16,), jnp.int32),
            pltpu.VMEM((16,), jnp.int32),
            pltpu.VMEM((16,), jnp.int32),
            pltpu.VMEM((16,), jnp.int32),
            pltpu.VMEM((APT // 64, 64), jnp.int32),
            pltpu.VMEM((APT // 64, 64), jnp.int32),
            pltpu.VMEM((64, D), jnp.float32),
            pltpu.SemaphoreType.DMA,
        ],
    )
    return f(eflat, x_flat)


def _combine_body(y_ref, pos2_ref, g_ref, out_ref,
                  posb_v, gb_v, pair0_v, pair1_v, ob_v, sem0, sem1):
    wid = lax.axis_index("s") * 2 + lax.axis_index("c")
    r0 = wid * 2
    pltpu.sync_copy(pos2_ref.at[pl.ds(r0, 2)], posb_v)
    pltpu.sync_copy(g_ref.at[pl.ds(r0, 2)], gb_v)
    pairs = (pair0_v, pair1_v)
    sems = (sem0, sem1)

    def start(c):
        idx = posb_v.at[c // 2, pl.ds((c % 2) * 32, 32)]
        return pltpu.async_copy(y_ref.at[idx], pairs[c % 2], sems[c % 2])

    cp = start(0)
    for c in range(4):
        nxt = start(c + 1) if c < 3 else None
        cp.wait()
        pair = pairs[c % 2]
        row = c // 2

        def body(i, _):
            g0 = plsc.load_gather(
                gb_v, [jnp.full((16,), row, jnp.int32),
                       jnp.broadcast_to((c % 2) * 32 + 2 * i, (16,))])
            g1 = plsc.load_gather(
                gb_v, [jnp.full((16,), row, jnp.int32),
                       jnp.broadcast_to((c % 2) * 32 + 2 * i + 1, (16,))])
            for ch in range(D // 16):
                va = pair[2 * i, pl.ds(ch * 16, 16)]
                vb = pair[2 * i + 1, pl.ds(ch * 16, 16)]
                ob_v[i, pl.ds(ch * 16, 16)] = va * g0 + vb * g1
            return 0

        lax.fori_loop(0, 16, body, 0)
        pltpu.sync_copy(ob_v, out_ref.at[pl.ds(wid * TPW + c * 16, 16)])
        cp = nxt


def _combine(y, pos2, gflat2d):
    mesh = plsc.VectorSubcoreMesh(core_axis_name="c", subcore_axis_name="s")
    f = pl.kernel(
        _combine_body,
        out_type=jax.ShapeDtypeStruct((T, D), jnp.float32),
        mesh=mesh,
        compiler_params=pltpu.CompilerParams(needs_layout_passes=False),
        scratch_types=[
            pltpu.VMEM((2, 64), jnp.int32),
            pltpu.VMEM((2, 64), jnp.float32),
            pltpu.VMEM((32, D), jnp.float32),
            pltpu.VMEM((32, D), jnp.float32),
            pltpu.VMEM((16, D), jnp.float32),
            pltpu.SemaphoreType.DMA,
            pltpu.SemaphoreType.DMA,
        ],
    )
    return f(y, pos2, gflat2d)


def _ffn_body(e_of_ref, sz_ref, xs_ref, w1_ref, b1_ref, w2_ref, b2_ref,
              y_ref, acc_ref):
    j = pl.program_id(1)

    @pl.when(j == 0)
    def _():
        acc_ref[...] = jnp.zeros_like(acc_ref)

    @pl.when(sz_ref[pl.program_id(0)] > 0)
    def _():
        h = jnp.dot(xs_ref[...], w1_ref[0], preferred_element_type=jnp.float32)
        h = jnp.maximum(h + b1_ref[0], 0.0)
        acc_ref[...] += jnp.dot(h, w2_ref[0], preferred_element_type=jnp.float32)

    @pl.when(j == NH - 1)
    def _():
        y_ref[...] = acc_ref[...] + b2_ref[0]


def _serp(i, j):
    return jnp.where(i % 2 == 0, j, NH - 1 - j)


def _ffn(e_of, sz, xs, w1, b1, w2, b2):
    grid_spec = pltpu.PrefetchScalarGridSpec(
        num_scalar_prefetch=2,
        grid=(NUM_BLOCKS, NH),
        in_specs=[
            pl.BlockSpec((BLK, D), lambda i, j, eo, sz: (i, 0)),
            pl.BlockSpec((1, D, HT), lambda i, j, eo, sz: (eo[i], 0, _serp(i, j))),
            pl.BlockSpec((1, 1, HT), lambda i, j, eo, sz: (eo[i], 0, _serp(i, j))),
            pl.BlockSpec((1, HT, D), lambda i, j, eo, sz: (eo[i], _serp(i, j), 0)),
            pl.BlockSpec((1, 1, D), lambda i, j, eo, sz: (eo[i], 0, 0)),
        ],
        out_specs=pl.BlockSpec((BLK, D), lambda i, j, eo, sz: (i, 0)),
        scratch_shapes=[pltpu.VMEM((BLK, D), jnp.float32)],
    )
    return pl.pallas_call(
        _ffn_body,
        grid_spec=grid_spec,
        out_shape=jax.ShapeDtypeStruct((NP, D), jnp.float32),
        compiler_params=pltpu.CompilerParams(
            dimension_semantics=("arbitrary", "arbitrary"),
            vmem_limit_bytes=128 * 1024 * 1024,
        ),
    )(e_of, sz, xs, w1, b1, w2, b2)


def kernel(x, Wg, bg, W1, b1, W2, b2):
    B, S, d = x.shape
    x_flat = x.reshape(T, d)
    wg_pad = jnp.pad(Wg, ((0, 0), (0, 128 - E)))
    bg_pad = jnp.pad(bg, (0, 128 - E)).reshape(1, 128)

    idx128, g128, aux = _gating(x_flat, wg_pad, bg_pad)
    aux = aux[0, 0]
    eflat = idx128[:, :TOPK].reshape(A)
    gflat2d = g128[:, :TOPK].reshape(A // 64, 64)

    xs, pos2, e_of, sz = _routing(eflat, x_flat)
    y = _ffn(e_of, sz, xs, W1, b1.reshape(E, 1, H), W2, b2.reshape(E, 1, D))
    out = _combine(y, pos2, gflat2d).reshape(B, S, d)
    return out, aux
